# baseline (speedup 1.0000x reference)
import jax
import jax.numpy as jnp
from jax import lax
from jax.experimental import pallas as pl
from jax.experimental.pallas import tpu as pltpu

_B, _C, _H, _W = 64, 96, 64, 64
_BB = 2
_NC = 4
_CC = _C // _NC


def _body(lab_ref, *refs):
    xrefs = refs[:_NC]
    yrefs = refs[_NC:2 * _NC]
    out_ref = refs[2 * _NC]
    step = pl.program_id(0)
    rev = jnp.broadcast_to(
        (_W - 1) - lax.broadcasted_iota(jnp.int32, (_CC, _H, _W), 2),
        (_CC, _H, _W))

    def _g(v):
        return jnp.take_along_axis(v, rev, axis=2)

    for i in range(_BB):
        r = lab_ref[step * _BB + i]
        l2 = jnp.float32(0.0)
        kl = jnp.float32(0.0)
        for j in range(_NC):
            x = xrefs[j][i]
            y = yrefs[j][i]

            xt = jnp.swapaxes(x, 1, 2)
            a = jnp.where((r == 1) | (r == 2), xt, x)
            g = _g(a)
            b = jnp.where(r == 0, a, g)
            c = jnp.swapaxes(b, 1, 2)
            xr = jnp.where(r <= 1, b, c)
            yg = jnp.where(r == 2, _g(y), y)

            diff = xr - yg
            l2 = l2 + jnp.sum(diff * diff)
            kl = kl + jnp.sum(xr * jnp.log(xr / jnp.maximum(yg, 1e-9)))
        out_ref[0, i, 0] = l2
        out_ref[0, i, 1] = kl


def kernel(hp, hp_rot, label_rot):
    chunk_spec = [
        pl.BlockSpec((_BB, _CC, _H, _W),
                     (lambda j: (lambda b, lab: (b, j, 0, 0)))(j))
        for j in range(_NC)
    ]
    grid_spec = pltpu.PrefetchScalarGridSpec(
        num_scalar_prefetch=1,
        grid=(_B // _BB,),
        in_specs=chunk_spec + chunk_spec,
        out_specs=[
            pl.BlockSpec(memory_space=pltpu.SMEM, block_shape=(1, _BB, 2),
                         index_map=lambda b, lab: (b, 0, 0)),
        ],
    )
    out = pl.pallas_call(
        _body,
        grid_spec=grid_spec,
        out_shape=[
            jax.ShapeDtypeStruct((_B // _BB, _BB, 2), jnp.float32),
        ],
    )(label_rot.astype(jnp.int32), *([hp] * _NC), *([hp_rot] * _NC))[0]
    kl_s = out[:, :, 1].sum() / _B
    l2_s = out[:, :, 0].sum() / (_B * _C * _H * _W)
    return kl_s * 0.4 + l2_s * 0.6
